# baseline (speedup 1.0000x reference)
import functools

import jax
import jax.numpy as jnp
from jax import lax
from jax.experimental import pallas as pl
from jax.experimental.pallas import tpu as pltpu
from jax.experimental.pallas import tpu_sc as plsc

_N = 10000
_E = 160000
_HID = 256
_FG = 64
_NGRP = 4
_NC, _NS = 2, 16
_NW = _NC * _NS
_CHUNK = 128
_EPW = 5120
_E_PAD = _EPW * _NW
_NCHUNK = _EPW // _CHUNK
_WB = 632
_ACC = _WB * _NS
_LAST = _N - _WB * (_NS - 1)

_PREC = lax.Precision.HIGHEST


def _mm_body(x_ref, w_ref, b_ref, o_ref, *, act):
    acc = jnp.dot(x_ref[...], w_ref[...], preferred_element_type=jnp.float32,
                  precision=_PREC)
    acc = acc + b_ref[...]
    if act == "relu":
        acc = jnp.maximum(acc, 0.0)
    o_ref[...] = acc


def _mm(x, w, b, act=None, bm=512):
    m, k = x.shape
    n = w.shape[1]
    assert m % bm == 0
    return pl.pallas_call(
        functools.partial(_mm_body, act=act),
        grid=(m // bm,),
        in_specs=[
            pl.BlockSpec((bm, k), lambda i: (i, 0)),
            pl.BlockSpec((k, n), lambda i: (0, 0)),
            pl.BlockSpec((1, n), lambda i: (0, 0)),
        ],
        out_specs=pl.BlockSpec((bm, n), lambda i: (i, 0)),
        out_shape=jax.ShapeDtypeStruct((m, n), jnp.float32),
    )(x, w, b.reshape(1, n))


def _mm_grouped_body(x_ref, w_ref, b_ref, o_ref):
    acc = jnp.dot(x_ref[...], w_ref[0], preferred_element_type=jnp.float32,
                  precision=_PREC)
    o_ref[0] = acc + b_ref[0]


def _mm_grouped(x, w, b, bm=1024):
    m, k = x.shape
    assert m % bm == 0
    w4 = w.reshape(k, _NGRP, _FG).transpose(1, 0, 2)
    return pl.pallas_call(
        _mm_grouped_body,
        grid=(_NGRP, m // bm),
        in_specs=[
            pl.BlockSpec((bm, k), lambda g, i: (i, 0)),
            pl.BlockSpec((1, k, _FG), lambda g, i: (g, 0, 0)),
            pl.BlockSpec((1, 1, _FG), lambda g, i: (g, 0, 0)),
        ],
        out_specs=pl.BlockSpec((1, bm, _FG), lambda g, i: (g, i, 0)),
        out_shape=jax.ShapeDtypeStruct((_NGRP, m, _FG), jnp.float32),
    )(x, w4, b.reshape(_NGRP, 1, _FG))


def _sc_edge_body(g, dh, eh, bh, ce4, src, dst, zz,
                  xe, num2, den2,
                  srcb, dstb, dhb, ehb, bhb, ceb, sigb, xeb,
                  accn, accd, sem):
    cid = lax.axis_index("c")
    sid = lax.axis_index("s")
    wid = sid * _NC + cid

    pltpu.sync_copy(zz, accn.at[pl.ds(sid * _WB, _WB)])
    pltpu.sync_copy(zz, accd.at[pl.ds(sid * _WB, _WB)])
    plsc.subcore_barrier()

    estart = wid * _EPW

    def chunk_body(c, carry):
        base = pl.multiple_of(estart + c * _CHUNK, _CHUNK)
        pltpu.sync_copy(src.at[pl.ds(base, _CHUNK)], srcb)
        pltpu.sync_copy(dst.at[pl.ds(base, _CHUNK)], dstb.at[0])
        pltpu.async_copy(dh.at[srcb], dhb, sem).wait()
        pltpu.async_copy(eh.at[dstb.at[0]], ehb, sem).wait()
        pltpu.async_copy(bh.at[srcb], bhb, sem).wait()
        pltpu.sync_copy(ce4.at[g].at[pl.ds(base, _CHUNK)], ceb)

        def edge_body(k, carry2):
            for j in range(_FG // 16):
                sl = pl.ds(j * 16, 16)
                en = dhb[k, sl] + ehb[k, sl] + ceb[k, sl]
                sg = 1.0 / (1.0 + jnp.exp(-en))
                sigb[k, sl] = sg
                xeb[k, sl] = en
                bhb[k, sl] = sg * bhb[k, sl]
            return carry2

        lax.fori_loop(0, _CHUNK, edge_body, 0)

        pltpu.sync_copy(xeb, xe.at[pl.ds(base, _CHUNK)])
        pltpu.sync_copy(bhb, accn.at[dstb.at[0]], add=True)
        pltpu.sync_copy(sigb, accd.at[dstb.at[0]], add=True)
        return carry

    lax.fori_loop(0, _NCHUNK, chunk_body, 0)
    plsc.subcore_barrier()

    rb = sid * _WB
    pltpu.sync_copy(accn.at[pl.ds(rb, _LAST)], num2.at[cid].at[pl.ds(rb, _LAST)])
    pltpu.sync_copy(accd.at[pl.ds(rb, _LAST)], den2.at[cid].at[pl.ds(rb, _LAST)])

    @pl.when(sid < _NS - 1)
    def _():
        pltpu.sync_copy(accn.at[pl.ds(rb + _LAST, _WB - _LAST)],
                        num2.at[cid].at[pl.ds(rb + _LAST, _WB - _LAST)])
        pltpu.sync_copy(accd.at[pl.ds(rb + _LAST, _WB - _LAST)],
                        den2.at[cid].at[pl.ds(rb + _LAST, _WB - _LAST)])


def _sc_edge(g, dh, eh, bh, ce4, src, dst, zz):
    f = pl.kernel(
        functools.partial(_sc_edge_body, g),
        out_type=[
            jax.ShapeDtypeStruct((_E_PAD, _FG), jnp.float32),
            jax.ShapeDtypeStruct((_NC, _N, _FG), jnp.float32),
            jax.ShapeDtypeStruct((_NC, _N, _FG), jnp.float32),
        ],
        mesh=plsc.VectorSubcoreMesh(core_axis_name="c", subcore_axis_name="s"),
        compiler_params=pltpu.CompilerParams(use_tc_tiling_on_sc=False),
        scratch_types=[
            pltpu.VMEM((_CHUNK,), jnp.int32),
            pltpu.VMEM((1, _CHUNK), jnp.int32),
            pltpu.VMEM((_CHUNK, _FG), jnp.float32),
            pltpu.VMEM((_CHUNK, _FG), jnp.float32),
            pltpu.VMEM((_CHUNK, _FG), jnp.float32),
            pltpu.VMEM((_CHUNK, _FG), jnp.float32),
            pltpu.VMEM((_CHUNK, _FG), jnp.float32),
            pltpu.VMEM((_CHUNK, _FG), jnp.float32),
            pltpu.VMEM_SHARED((_ACC, _FG), jnp.float32),
            pltpu.VMEM_SHARED((_ACC, _FG), jnp.float32),
            pltpu.SemaphoreType.DMA,
        ],
    )
    return f(dh, eh, bh, ce4, src, dst, zz)


def _k3_body(ah, n0, n1, n2, n3, d0, d1, d2, d3, sn, xh, sums):
    num = jnp.concatenate([n0[0] + n0[1], n1[0] + n1[1],
                           n2[0] + n2[1], n3[0] + n3[1]], axis=1)
    den = jnp.concatenate([d0[0] + d0[1], d1[0] + d1[1],
                           d2[0] + d2[1], d3[0] + d3[1]], axis=1)
    x = (ah[...] + num / (den + 1e-6)) * sn[...]
    xh[...] = x
    contrib = jnp.concatenate([jnp.sum(x, axis=0, keepdims=True),
                               jnp.sum(x * x, axis=0, keepdims=True)], axis=0)
    i = pl.program_id(0)

    @pl.when(i == 0)
    def _():
        sums[...] = contrib

    @pl.when(i != 0)
    def _():
        sums[...] += contrib


def _k3(ah, nums, dens, sn, bm=400):
    grid = (_N // bm,)
    specs_nd = [pl.BlockSpec((_NC, bm, _FG), lambda i: (0, i, 0))] * (2 * _NGRP)
    return pl.pallas_call(
        _k3_body,
        grid=grid,
        in_specs=[pl.BlockSpec((bm, _HID), lambda i: (i, 0))] + specs_nd
        + [pl.BlockSpec((bm, 1), lambda i: (i, 0))],
        out_specs=[pl.BlockSpec((bm, _HID), lambda i: (i, 0)),
                   pl.BlockSpec((2, _HID), lambda i: (0, 0))],
        out_shape=[jax.ShapeDtypeStruct((_N, _HID), jnp.float32),
                   jax.ShapeDtypeStruct((2, _HID), jnp.float32)],
    )(ah, *nums, *dens, sn)


def _bn_apply_body(x, sums, gm, bt, res, o, *, count):
    mu = sums[0:1] / count
    var = sums[1:2] / count - mu * mu
    rstd = lax.rsqrt(var + 1e-5)
    y = gm[...] * (x[...] - mu) * rstd + bt[...]
    o[...] = res[...] + jnp.maximum(y, 0.0)


def _bn_apply(x, sums, gm, bt, res, count, bm):
    m = x.shape[0]
    return pl.pallas_call(
        functools.partial(_bn_apply_body, count=count),
        grid=(m // bm,),
        in_specs=[
            pl.BlockSpec((bm, _HID), lambda i: (i, 0)),
            pl.BlockSpec((2, _HID), lambda i: (0, 0)),
            pl.BlockSpec((1, _HID), lambda i: (0, 0)),
            pl.BlockSpec((1, _HID), lambda i: (0, 0)),
            pl.BlockSpec((bm, _HID), lambda i: (i, 0)),
        ],
        out_specs=pl.BlockSpec((bm, _HID), lambda i: (i, 0)),
        out_shape=jax.ShapeDtypeStruct((m, _HID), jnp.float32),
    )(x, sums, gm.reshape(1, _HID), bt.reshape(1, _HID), res)


def _e_stats_body(x0, x1, x2, x3, sn, sums):
    x = jnp.concatenate([x0[...], x1[...], x2[...], x3[...]], axis=1) * sn[...]
    contrib = jnp.concatenate([jnp.sum(x, axis=0, keepdims=True),
                               jnp.sum(x * x, axis=0, keepdims=True)], axis=0)
    i = pl.program_id(0)

    @pl.when(i == 0)
    def _():
        sums[...] = contrib

    @pl.when(i != 0)
    def _():
        sums[...] += contrib


def _e_stats(xes, sn, bm=640):
    return pl.pallas_call(
        _e_stats_body,
        grid=(_E // bm,),
        in_specs=[pl.BlockSpec((bm, _FG), lambda i: (i, 0))] * _NGRP
        + [pl.BlockSpec((bm, 1), lambda i: (i, 0))],
        out_specs=pl.BlockSpec((2, _HID), lambda i: (0, 0)),
        out_shape=jax.ShapeDtypeStruct((2, _HID), jnp.float32),
    )(*xes, sn)


def _e_bn_body(x0, x1, x2, x3, sn, sums, gm, bt, res, o):
    x = jnp.concatenate([x0[...], x1[...], x2[...], x3[...]], axis=1) * sn[...]
    mu = sums[0:1] / float(_E)
    var = sums[1:2] / float(_E) - mu * mu
    rstd = lax.rsqrt(var + 1e-5)
    y = gm[...] * (x - mu) * rstd + bt[...]
    o[...] = res[...] + jnp.maximum(y, 0.0)


def _e_bn(xes, sn, sums, gm, bt, res, bm=1024):
    return pl.pallas_call(
        _e_bn_body,
        grid=(_E_PAD // bm,),
        in_specs=[pl.BlockSpec((bm, _FG), lambda i: (i, 0))] * _NGRP
        + [
            pl.BlockSpec((bm, 1), lambda i: (i, 0)),
            pl.BlockSpec((2, _HID), lambda i: (0, 0)),
            pl.BlockSpec((1, _HID), lambda i: (0, 0)),
            pl.BlockSpec((1, _HID), lambda i: (0, 0)),
            pl.BlockSpec((bm, _HID), lambda i: (i, 0)),
        ],
        out_specs=pl.BlockSpec((bm, _HID), lambda i: (i, 0)),
        out_shape=jax.ShapeDtypeStruct((_E_PAD, _HID), jnp.float32),
    )(*xes, sn, sums, gm.reshape(1, _HID), bt.reshape(1, _HID), res)


def _mlp_body(x, w1, b1, w2, b2, w3, b3, o):
    y = jnp.maximum(jnp.dot(x[...], w1[...], preferred_element_type=jnp.float32,
                            precision=_PREC) + b1[...], 0.0)
    y = jnp.maximum(jnp.dot(y, w2[...], preferred_element_type=jnp.float32,
                            precision=_PREC) + b2[...], 0.0)
    z = jnp.dot(y, w3[...], preferred_element_type=jnp.float32,
                precision=_PREC) + b3[...]
    o[...] = 1.0 / (1.0 + jnp.exp(-z))


def _mlp(x, mlp_params, bm=400):
    w1, b1 = mlp_params[0]["w"], mlp_params[0]["b"]
    w2, b2 = mlp_params[1]["w"], mlp_params[1]["b"]
    w3, b3 = mlp_params[2]["w"], mlp_params[2]["b"]
    w3p = jnp.pad(w3, ((0, 0), (0, 127)))
    b3p = jnp.pad(b3, ((0, 127)))
    out = pl.pallas_call(
        _mlp_body,
        grid=(_N // bm,),
        in_specs=[
            pl.BlockSpec((bm, _HID), lambda i: (i, 0)),
            pl.BlockSpec((_HID, 128), lambda i: (0, 0)),
            pl.BlockSpec((1, 128), lambda i: (0, 0)),
            pl.BlockSpec((128, 64), lambda i: (0, 0)),
            pl.BlockSpec((1, 64), lambda i: (0, 0)),
            pl.BlockSpec((64, 128), lambda i: (0, 0)),
            pl.BlockSpec((1, 128), lambda i: (0, 0)),
        ],
        out_specs=pl.BlockSpec((bm, 128), lambda i: (i, 0)),
        out_shape=jax.ShapeDtypeStruct((_N, 128), jnp.float32),
    )(x, w1, b1.reshape(1, 128), w2, b2.reshape(1, 64), w3p, b3p.reshape(1, 128))
    return out[:, :1]


def kernel(h, e, snorm_n, snorm_e, edge_index, params):
    npad = _E_PAD - _E
    src = jnp.concatenate(
        [edge_index[0], jnp.arange(npad, dtype=jnp.int32) % _N])
    dst = jnp.concatenate(
        [edge_index[1], _N + jnp.arange(npad, dtype=jnp.int32) % (_ACC - _N)])
    sn_e = jnp.concatenate(
        [snorm_e, jnp.zeros((npad, 1), jnp.float32)], axis=0)
    e_pad = jnp.concatenate([e, jnp.zeros((npad, e.shape[1]), jnp.float32)])
    zz = jnp.zeros((_WB, _FG), jnp.float32)

    hh = _mm(h, params["emb_h"]["w"], params["emb_h"]["b"], bm=400)
    ee = _mm(e_pad, params["emb_e"]["w"], params["emb_e"]["b"], bm=1024)

    for lp in params["layers"]:
        wcat = jnp.concatenate(
            [lp["A"]["w"], lp["B"]["w"], lp["D"]["w"], lp["E"]["w"]], axis=1)
        bcat = jnp.concatenate(
            [lp["A"]["b"], lp["B"]["b"], lp["D"]["b"], lp["E"]["b"]])
        hcat = _mm(hh, wcat, bcat, bm=400)
        ah = hcat[:, 0:_HID]
        ce4 = _mm_grouped(ee, lp["C"]["w"], lp["C"]["b"], bm=1024)

        xes, nums, dens = [], [], []
        for g in range(_NGRP):
            f0 = g * _FG
            bh_g = hcat[:, _HID + f0:_HID + f0 + _FG]
            dh_g = hcat[:, 2 * _HID + f0:2 * _HID + f0 + _FG]
            eh_g = hcat[:, 3 * _HID + f0:3 * _HID + f0 + _FG]
            xe_g, num2, den2 = _sc_edge(g, dh_g, eh_g, bh_g, ce4,
                                        src, dst, zz)
            xes.append(xe_g)
            nums.append(num2)
            dens.append(den2)

        xh, hsums = _k3(ah, nums, dens, snorm_n)
        hh = _bn_apply(xh, hsums, lp["bnh_g"], lp["bnh_b"], hh,
                       float(_N), bm=400)
        esums = _e_stats(xes, sn_e)
        ee = _e_bn(xes, sn_e, esums, lp["bne_g"], lp["bne_b"], ee, bm=1024)

    return _mlp(hh, params["mlp"])
